# baseline (speedup 1.0000x reference)
import jax
import jax.numpy as jnp
from jax.experimental import pallas as pl
from jax.experimental.pallas import tpu as pltpu

_SLOPE = (1.0 / 8.0 + 1.0 / 3.0) / 2.0


def _rrelu_block(x_ref, o_ref):
    x = x_ref[...]
    o_ref[...] = jnp.where(x >= 0, x, x * _SLOPE)


def kernel(node_feats, edge_feats, edge_index, W_neigh, b_neigh, W_loop, b_loop):
    n, d = node_feats.shape
    block = 10000
    return pl.pallas_call(
        _rrelu_block,
        grid=(n // block,),
        in_specs=[pl.BlockSpec((block, d), lambda i: (i, 0))],
        out_specs=pl.BlockSpec((block, d), lambda i: (i, 0)),
        out_shape=jax.ShapeDtypeStruct((n, d), node_feats.dtype),
        compiler_params=pltpu.CompilerParams(
            dimension_semantics=("parallel",),
        ),
    )(node_feats)


# speedup vs baseline: 1.2065x; 1.2065x over previous
import jax
import jax.numpy as jnp
from jax.experimental import pallas as pl
from jax.experimental.pallas import tpu as pltpu

_SLOPE = (1.0 / 8.0 + 1.0 / 3.0) / 2.0


def _rrelu_block(x_ref, o_ref):
    x = x_ref[...]
    o_ref[...] = jnp.where(x >= 0, x, x * _SLOPE)


def kernel(node_feats, edge_feats, edge_index, W_neigh, b_neigh, W_loop, b_loop):
    n, d = node_feats.shape
    block = 5000
    return pl.pallas_call(
        _rrelu_block,
        grid=(n // block,),
        in_specs=[pl.BlockSpec((block, d), lambda i: (i, 0))],
        out_specs=pl.BlockSpec((block, d), lambda i: (i, 0)),
        out_shape=jax.ShapeDtypeStruct((n, d), node_feats.dtype),
        compiler_params=pltpu.CompilerParams(
            dimension_semantics=("arbitrary",),
        ),
    )(node_feats)
